# baseline (speedup 1.0000x reference)
import jax
import jax.numpy as jnp
from jax.experimental import pallas as pl

BATCH = 64
IN_FEATURES = 512
OUT_FEATURES = 100000
BLOCK_N = 11264
HALF_N = BLOCK_N // 2


def _row_ssq(w):
    return (w[:, 0:128] * w[:, 0:128] + w[:, 128:256] * w[:, 128:256]
            + w[:, 256:384] * w[:, 256:384] + w[:, 384:512] * w[:, 384:512])


def _fused_norm_matmul_kernel(x_ref, wa_ref, wb_ref, out_ref):
    x = x_ref[...]
    ones = jnp.ones((1, 128), dtype=jnp.float32)
    for half, w_ref in enumerate((wa_ref, wb_ref)):
        w = w_ref[...]
        s = _row_ssq(w)
        ssq = jax.lax.dot_general(
            ones, s,
            dimension_numbers=(((1,), (1,)), ((), ())),
            preferred_element_type=jnp.float32,
        )
        inv = 1.0 / jnp.maximum(jnp.sqrt(ssq), 1e-12)
        acc = jax.lax.dot_general(
            x, w,
            dimension_numbers=(((1,), (1,)), ((), ())),
            preferred_element_type=jnp.float32,
        )
        out_ref[:, half * HALF_N:(half + 1) * HALF_N] = acc * inv


def kernel(x, global_label, weight):
    del global_label
    grid = pl.cdiv(OUT_FEATURES, BLOCK_N)
    return pl.pallas_call(
        _fused_norm_matmul_kernel,
        grid=(grid,),
        in_specs=[
            pl.BlockSpec((BATCH, IN_FEATURES), lambda i: (0, 0)),
            pl.BlockSpec((HALF_N, IN_FEATURES), lambda i: (2 * i, 0)),
            pl.BlockSpec((HALF_N, IN_FEATURES), lambda i: (2 * i + 1, 0)),
        ],
        out_specs=pl.BlockSpec((BATCH, BLOCK_N), lambda i: (0, i)),
        out_shape=jax.ShapeDtypeStruct((BATCH, OUT_FEATURES), jnp.float32),
    )(x, weight, weight)


# speedup vs baseline: 1.0070x; 1.0070x over previous
import jax
import jax.numpy as jnp
from jax.experimental import pallas as pl

BATCH = 64
IN_FEATURES = 512
OUT_FEATURES = 100000
BLOCK_N = 9216
HALF_N = BLOCK_N // 2


def _row_ssq(w):
    return (w[:, 0:128] * w[:, 0:128] + w[:, 128:256] * w[:, 128:256]
            + w[:, 256:384] * w[:, 256:384] + w[:, 384:512] * w[:, 384:512])


def _fused_norm_matmul_kernel(x_ref, wa_ref, wb_ref, out_ref):
    x = x_ref[...]
    ones = jnp.ones((1, 128), dtype=jnp.float32)
    for half, w_ref in enumerate((wa_ref, wb_ref)):
        w = w_ref[...]
        s = _row_ssq(w)
        ssq = jax.lax.dot_general(
            ones, s,
            dimension_numbers=(((1,), (1,)), ((), ())),
            preferred_element_type=jnp.float32,
        )
        inv = 1.0 / jnp.maximum(jnp.sqrt(ssq), 1e-12)
        acc = jax.lax.dot_general(
            x, w,
            dimension_numbers=(((1,), (1,)), ((), ())),
            preferred_element_type=jnp.float32,
        )
        out_ref[:, half * HALF_N:(half + 1) * HALF_N] = acc * inv


def kernel(x, global_label, weight):
    del global_label
    grid = pl.cdiv(OUT_FEATURES, BLOCK_N)
    return pl.pallas_call(
        _fused_norm_matmul_kernel,
        grid=(grid,),
        in_specs=[
            pl.BlockSpec((BATCH, IN_FEATURES), lambda i: (0, 0)),
            pl.BlockSpec((HALF_N, IN_FEATURES), lambda i: (2 * i, 0)),
            pl.BlockSpec((HALF_N, IN_FEATURES), lambda i: (2 * i + 1, 0)),
        ],
        out_specs=pl.BlockSpec((BATCH, BLOCK_N), lambda i: (0, i)),
        out_shape=jax.ShapeDtypeStruct((BATCH, OUT_FEATURES), jnp.float32),
    )(x, weight, weight)


# speedup vs baseline: 1.0117x; 1.0046x over previous
import jax
import jax.numpy as jnp
from jax.experimental import pallas as pl

BATCH = 64
IN_FEATURES = 512
OUT_FEATURES = 100000
BLOCK_N = 8448
HALF_N = BLOCK_N // 2


def _row_ssq(w):
    return (w[:, 0:128] * w[:, 0:128] + w[:, 128:256] * w[:, 128:256]
            + w[:, 256:384] * w[:, 256:384] + w[:, 384:512] * w[:, 384:512])


def _fused_norm_matmul_kernel(x_ref, wa_ref, wb_ref, out_ref):
    x = x_ref[...]
    ones = jnp.ones((1, 128), dtype=jnp.float32)
    for half, w_ref in enumerate((wa_ref, wb_ref)):
        w = w_ref[...]
        s = _row_ssq(w)
        ssq = jax.lax.dot_general(
            ones, s,
            dimension_numbers=(((1,), (1,)), ((), ())),
            preferred_element_type=jnp.float32,
        )
        inv = 1.0 / jnp.maximum(jnp.sqrt(ssq), 1e-12)
        acc = jax.lax.dot_general(
            x, w,
            dimension_numbers=(((1,), (1,)), ((), ())),
            preferred_element_type=jnp.float32,
        )
        out_ref[:, half * HALF_N:(half + 1) * HALF_N] = acc * inv


def kernel(x, global_label, weight):
    del global_label
    grid = pl.cdiv(OUT_FEATURES, BLOCK_N)
    return pl.pallas_call(
        _fused_norm_matmul_kernel,
        grid=(grid,),
        in_specs=[
            pl.BlockSpec((BATCH, IN_FEATURES), lambda i: (0, 0)),
            pl.BlockSpec((HALF_N, IN_FEATURES), lambda i: (2 * i, 0)),
            pl.BlockSpec((HALF_N, IN_FEATURES), lambda i: (2 * i + 1, 0)),
        ],
        out_specs=pl.BlockSpec((BATCH, BLOCK_N), lambda i: (0, i)),
        out_shape=jax.ShapeDtypeStruct((BATCH, OUT_FEATURES), jnp.float32),
    )(x, weight, weight)
